# Initial kernel scaffold; baseline (speedup 1.0000x reference)
import jax
import jax.numpy as jnp
from jax.experimental import pallas as pl


def kernel(x_nchw, weight, bias, gamma, beta):
    raise NotImplementedError("write your pallas kernel here")


# speedup vs baseline: 3.4228x; 3.4228x over previous
import functools

import jax
import jax.numpy as jnp
from jax import lax
from jax.experimental import pallas as pl
from jax.experimental.pallas import tpu as pltpu


def _conv_stats_kernel(x_ref, a_ref, bias_ref, y_ref, stat_ref, *, W, Cin):
    x = x_ref[0]
    P = x.shape[1]
    xb = x.astype(jnp.bfloat16)
    wcol = lax.broadcasted_iota(jnp.int32, (1, P), 1) % W

    slabs = []
    for dh in (-1, 0, 1):
        for dw in (-1, 0, 1):
            s = dh * W + dw
            if s > 0:
                sh = jnp.concatenate(
                    [xb[:, s:], jnp.zeros((Cin, s), jnp.bfloat16)], axis=1)
            elif s < 0:
                sh = jnp.concatenate(
                    [jnp.zeros((Cin, -s), jnp.bfloat16), xb[:, :P + s]], axis=1)
            else:
                sh = xb
            if dw == -1:
                sh = jnp.where(wcol >= 1, sh, jnp.bfloat16(0))
            elif dw == 1:
                sh = jnp.where(wcol < W - 1, sh, jnp.bfloat16(0))
            slabs.append(sh)
    bmat = jnp.concatenate(slabs, axis=0)

    acc = lax.dot_general(a_ref[...], bmat, (((1,), (0,)), ((), ())),
                          preferred_element_type=jnp.float32)
    acc = acc + bias_ref[...]
    y_ref[0] = acc

    s1 = jnp.sum(acc, axis=1, keepdims=True)
    s2 = jnp.sum(acc * acc, axis=1, keepdims=True)
    stat_ref[0] = jnp.concatenate([s1, s2], axis=1)


def _bn_lrelu_kernel(y_ref, scale_ref, shift_ref, o_ref):
    v = y_ref[...] * scale_ref[...] + shift_ref[...]
    o_ref[...] = jnp.where(v >= 0, v, 0.01 * v)


def kernel(x_nchw, weight, bias, gamma, beta, eps=1e-5):
    N, Cin, H, W = x_nchw.shape
    Cout = weight.shape[0]
    P = H * W

    xflat = x_nchw.reshape(N, Cin, P)
    a_mat = jnp.transpose(weight, (0, 2, 3, 1)).reshape(
        Cout, 9 * Cin).astype(jnp.bfloat16)
    bias_col = bias.astype(jnp.float32).reshape(Cout, 1)

    cparams = pltpu.CompilerParams(dimension_semantics=("parallel",))

    y, stats = pl.pallas_call(
        functools.partial(_conv_stats_kernel, W=W, Cin=Cin),
        out_shape=(jax.ShapeDtypeStruct((N, Cout, P), jnp.float32),
                   jax.ShapeDtypeStruct((N, Cout, 2), jnp.float32)),
        grid=(N,),
        in_specs=[pl.BlockSpec((1, Cin, P), lambda n: (n, 0, 0)),
                  pl.BlockSpec((Cout, 9 * Cin), lambda n: (0, 0)),
                  pl.BlockSpec((Cout, 1), lambda n: (0, 0))],
        out_specs=(pl.BlockSpec((1, Cout, P), lambda n: (n, 0, 0)),
                   pl.BlockSpec((1, Cout, 2), lambda n: (n, 0, 0))),
        compiler_params=cparams,
    )(xflat, a_mat, bias_col)

    st = stats.sum(axis=0)
    cnt = jnp.float32(N * P)
    mean = st[:, 0] / cnt
    var = jnp.maximum(st[:, 1] / cnt - mean * mean, 0.0)
    scale = gamma.astype(jnp.float32) * lax.rsqrt(var + eps)
    shift = beta.astype(jnp.float32) - mean * scale

    nb = 4 if N % 4 == 0 else 1
    out = pl.pallas_call(
        _bn_lrelu_kernel,
        out_shape=jax.ShapeDtypeStruct((N, Cout, P), jnp.float32),
        grid=(N // nb,),
        in_specs=[pl.BlockSpec((nb, Cout, P), lambda n: (n, 0, 0)),
                  pl.BlockSpec((Cout, 1), lambda n: (0, 0)),
                  pl.BlockSpec((Cout, 1), lambda n: (0, 0))],
        out_specs=pl.BlockSpec((nb, Cout, P), lambda n: (n, 0, 0)),
        compiler_params=cparams,
    )(y, scale.reshape(Cout, 1), shift.reshape(Cout, 1))

    return out.reshape(N, Cout, H, W)
